# baseline (speedup 1.0000x reference)
import jax
import jax.numpy as jnp
from jax.experimental import pallas as pl
from jax.experimental.pallas import tpu as pltpu

_BV = 3200


def _medusa_body(x_ref, w1_ref, b1_ref, w2_ref, out_ref, h_ref):
    j = pl.program_id(1)

    @pl.when(j == 0)
    def _compute_hidden():
        h = jax.lax.dot_general(
            x_ref[...], w1_ref[0],
            dimension_numbers=(((1,), (1,)), ((), ())),
            preferred_element_type=jnp.float32,
        ) + b1_ref[0]
        h_ref[...] = h * jax.nn.sigmoid(h)

    out_ref[0] = jax.lax.dot_general(
        h_ref[...], w2_ref[0],
        dimension_numbers=(((1,), (1,)), ((), ())),
        preferred_element_type=jnp.float32,
    )


def kernel(hidden_states, W1, b1, W2):
    B, S, H = hidden_states.shape
    NH, V, _ = W2.shape
    x = hidden_states.reshape(B * S, H)
    b1r = b1.reshape(NH, 1, H)

    out = pl.pallas_call(
        _medusa_body,
        grid=(NH, V // _BV),
        in_specs=[
            pl.BlockSpec((B * S, H), lambda i, j: (0, 0)),
            pl.BlockSpec((1, H, H), lambda i, j: (i, 0, 0)),
            pl.BlockSpec((1, 1, H), lambda i, j: (i, 0, 0)),
            pl.BlockSpec((1, _BV, H), lambda i, j: (i, j, 0)),
        ],
        out_specs=pl.BlockSpec((1, B * S, _BV), lambda i, j: (i, 0, j)),
        out_shape=jax.ShapeDtypeStruct((NH, B * S, V), jnp.float32),
        scratch_shapes=[pltpu.VMEM((B * S, H), jnp.float32)],
        compiler_params=pltpu.CompilerParams(
            dimension_semantics=("arbitrary", "arbitrary"),
        ),
    )(x, W1, b1r, W2)

    return tuple(out[i].reshape(B, S, V) for i in range(NH))


# speedup vs baseline: 1.0015x; 1.0015x over previous
import jax
import jax.numpy as jnp
from jax.experimental import pallas as pl
from jax.experimental.pallas import tpu as pltpu

_BV = 3200


def _medusa_body(x_ref, w1_ref, b1_ref, w2_ref, out_ref, h_ref):
    j = pl.program_id(1)

    @pl.when(j == 0)
    def _compute_hidden():
        h = jax.lax.dot_general(
            x_ref[...], w1_ref[0],
            dimension_numbers=(((1,), (1,)), ((), ())),
            preferred_element_type=jnp.float32,
        ) + b1_ref[0]
        h_ref[...] = h * jax.nn.sigmoid(h)

    out_ref[0] = jax.lax.dot_general(
        h_ref[...], w2_ref[0],
        dimension_numbers=(((1,), (1,)), ((), ())),
        preferred_element_type=jnp.float32,
    )


def kernel(hidden_states, W1, b1, W2):
    B, S, H = hidden_states.shape
    NH, V, _ = W2.shape
    x = hidden_states.reshape(B * S, H)
    b1r = b1.reshape(NH, 1, H)

    out = pl.pallas_call(
        _medusa_body,
        grid=(NH, V // _BV),
        in_specs=[
            pl.BlockSpec((B * S, H), lambda i, j: (0, 0)),
            pl.BlockSpec((1, H, H), lambda i, j: (i, 0, 0)),
            pl.BlockSpec((1, 1, H), lambda i, j: (i, 0, 0)),
            pl.BlockSpec((1, _BV, H), lambda i, j: (i, j, 0)),
        ],
        out_specs=pl.BlockSpec((1, B * S, _BV), lambda i, j: (i, 0, j)),
        out_shape=jax.ShapeDtypeStruct((NH, B * S, V), jnp.float32),
        scratch_shapes=[pltpu.VMEM((B * S, H), jnp.float32)],
        compiler_params=pltpu.CompilerParams(
            dimension_semantics=("parallel", "arbitrary"),
        ),
    )(x, W1, b1r, W2)

    return tuple(out[i].reshape(B, S, V) for i in range(NH))


# speedup vs baseline: 1.0615x; 1.0599x over previous
import jax
import jax.numpy as jnp
from jax.experimental import pallas as pl
from jax.experimental.pallas import tpu as pltpu

_BV = 640


def _medusa_body(x_ref, w1_ref, b1_ref, w2_ref, o0, o1, o2, o3, h_ref):
    j = pl.program_id(0)

    @pl.when(j == 0)
    def _compute_hidden():
        for k in range(4):
            h = jax.lax.dot_general(
                x_ref[...], w1_ref[k],
                dimension_numbers=(((1,), (1,)), ((), ())),
                preferred_element_type=jnp.float32,
            ) + b1_ref[k]
            h_ref[k] = h * jax.nn.sigmoid(h)

    for k, o in enumerate((o0, o1, o2, o3)):
        o[...] = jax.lax.dot_general(
            h_ref[k], w2_ref[k],
            dimension_numbers=(((1,), (1,)), ((), ())),
            preferred_element_type=jnp.float32,
        )


def kernel(hidden_states, W1, b1, W2):
    B, S, H = hidden_states.shape
    NH, V, _ = W2.shape
    x = hidden_states.reshape(B * S, H)

    outs = pl.pallas_call(
        _medusa_body,
        grid=(V // _BV,),
        in_specs=[
            pl.BlockSpec((B * S, H), lambda j: (0, 0)),
            pl.BlockSpec((NH, H, H), lambda j: (0, 0, 0)),
            pl.BlockSpec((NH, H), lambda j: (0, 0)),
            pl.BlockSpec((NH, _BV, H), lambda j: (0, j, 0)),
        ],
        out_specs=[pl.BlockSpec((B * S, _BV), lambda j: (0, j))
                   for _ in range(NH)],
        out_shape=[jax.ShapeDtypeStruct((B * S, V), jnp.float32)
                   for _ in range(NH)],
        scratch_shapes=[pltpu.VMEM((NH, B * S, H), jnp.float32)],
        compiler_params=pltpu.CompilerParams(
            dimension_semantics=("arbitrary",),
            vmem_limit_bytes=128 * 1024 * 1024,
        ),
    )(x, W1, b1, W2)

    return tuple(o.reshape(B, S, V) for o in outs)
